# Initial kernel scaffold; baseline (speedup 1.0000x reference)
import functools

import jax
import jax.numpy as jnp
from jax.experimental import pallas as pl
from jax.experimental.pallas import tpu as pltpu

N = 10000
E = 320000
D_IN = 128
D_OUT = 256


def _leaky(x, slope):
    return jnp.where(x >= 0, x, slope * x)


def _front_body(x_ref, w_ref, as_ref, ad_ref, h_ref, asn_ref, adn_ref):
    h = jax.lax.dot_general(x_ref[...], w_ref[...], (((1,), (1,)), ((), ())),
                            preferred_element_type=jnp.float32)
    h_ref[...] = h
    asn_ref[...] = jax.lax.dot_general(h, as_ref[...], (((1,), (0,)), ((), ())),
                                       preferred_element_type=jnp.float32)
    adn_ref[...] = jax.lax.dot_general(h, ad_ref[...], (((1,), (0,)), ((), ())),
                                       preferred_element_type=jnp.float32)


def _front(x, W, att_src, att_dst):
    return pl.pallas_call(
        _front_body,
        out_shape=(
            jax.ShapeDtypeStruct((N, D_OUT), jnp.float32),
            jax.ShapeDtypeStruct((N, 1), jnp.float32),
            jax.ShapeDtypeStruct((N, 1), jnp.float32),
        ),
    )(x, W, att_src.reshape(D_OUT, 1), att_dst.reshape(D_OUT, 1))


def _mid_body(agg_ref, den_ref, h_ref, as_ref, ad_ref, b_ref, u_ref, s_ref):
    w_self = jnp.exp(_leaky(as_ref[...] + ad_ref[...], 0.2))
    agg = agg_ref[...] + w_self * h_ref[...]
    den = den_ref[...] + w_self
    u = _leaky(agg / (den + 1e-16) + b_ref[...], 0.02)
    u_ref[...] = u
    s_ref[0] = jnp.sum(u * u)


def _mid(agg, denom, h, a_s, a_d, bias):
    return pl.pallas_call(
        _mid_body,
        out_shape=(
            jax.ShapeDtypeStruct((N, D_OUT), jnp.float32),
            jax.ShapeDtypeStruct((1,), jnp.float32),
        ),
        out_specs=(
            pl.BlockSpec(memory_space=pltpu.VMEM),
            pl.BlockSpec(memory_space=pltpu.SMEM),
        ),
    )(agg, denom, h, a_s, a_d, bias.reshape(1, D_OUT))


def _bigmm_body(s_ref, ui_ref, uj_ref, o_ref):
    inv = 1.0 / s_ref[0]
    t = jax.lax.dot_general(ui_ref[...], uj_ref[...], (((1,), (1,)), ((), ())),
                            preferred_element_type=jnp.float32)
    o_ref[...] = jax.nn.sigmoid(t * inv)


def _bigmm(u, s, bi=1000, bj=2000):
    grid = (N // bi, N // bj)
    return pl.pallas_call(
        _bigmm_body,
        grid=grid,
        in_specs=[
            pl.BlockSpec(memory_space=pltpu.SMEM),
            pl.BlockSpec((bi, D_OUT), lambda i, j: (i, 0)),
            pl.BlockSpec((bj, D_OUT), lambda i, j: (j, 0)),
        ],
        out_specs=pl.BlockSpec((bi, bj), lambda i, j: (i, j)),
        out_shape=jax.ShapeDtypeStruct((N, N), jnp.float32),
    )(s, u, u)


def kernel(x, edge_index, W, att_src, att_dst, bias):
    h, a_s, a_d = _front(x, W, att_src, att_dst)
    src = edge_index[0]
    dst = edge_index[1]
    asn = a_s[:, 0]
    adn = a_d[:, 0]
    w = jnp.exp(_leaky(asn[src] + adn[dst], 0.2))
    denom = jax.ops.segment_sum(w, dst, num_segments=N)
    agg = jax.ops.segment_sum(w[:, None] * h[src], dst, num_segments=N)
    u, s = _mid(agg, denom.reshape(N, 1), h, a_s, a_d, bias)
    return _bigmm(u, s)


# speedup vs baseline: 1.8075x; 1.8075x over previous
import functools

import jax
import jax.numpy as jnp
from jax.experimental import pallas as pl
from jax.experimental.pallas import tpu as pltpu

N = 10000
E = 320000
D_IN = 128
D_OUT = 256


def _leaky(x, slope):
    return jnp.where(x >= 0, x, slope * x)


def _front_body(x_ref, w_ref, as_ref, ad_ref, h_ref, asn_ref, adn_ref):
    h = jax.lax.dot_general(x_ref[...], w_ref[...], (((1,), (1,)), ((), ())),
                            preferred_element_type=jnp.float32)
    h_ref[...] = h
    asn_ref[...] = jax.lax.dot_general(h, as_ref[...], (((1,), (0,)), ((), ())),
                                       preferred_element_type=jnp.float32)
    adn_ref[...] = jax.lax.dot_general(h, ad_ref[...], (((1,), (0,)), ((), ())),
                                       preferred_element_type=jnp.float32)


def _front(x, W, att_src, att_dst):
    return pl.pallas_call(
        _front_body,
        out_shape=(
            jax.ShapeDtypeStruct((N, D_OUT), jnp.float32),
            jax.ShapeDtypeStruct((N, 1), jnp.float32),
            jax.ShapeDtypeStruct((N, 1), jnp.float32),
        ),
    )(x, W, att_src.reshape(D_OUT, 1), att_dst.reshape(D_OUT, 1))


def _mid_body(agg_ref, den_ref, h_ref, as_ref, ad_ref, b_ref, u_ref, s_ref):
    w_self = jnp.exp(_leaky(as_ref[...] + ad_ref[...], 0.2))
    agg = agg_ref[...] + w_self * h_ref[...]
    den = den_ref[...] + w_self
    u = _leaky(agg / (den + 1e-16) + b_ref[...], 0.02)
    u_ref[...] = u
    s_ref[0] = jnp.sum(u * u)


def _mid(agg, denom, h, a_s, a_d, bias):
    return pl.pallas_call(
        _mid_body,
        out_shape=(
            jax.ShapeDtypeStruct((N, D_OUT), jnp.float32),
            jax.ShapeDtypeStruct((1,), jnp.float32),
        ),
        out_specs=(
            pl.BlockSpec(memory_space=pltpu.VMEM),
            pl.BlockSpec(memory_space=pltpu.SMEM),
        ),
    )(agg, denom, h, a_s, a_d, bias.reshape(1, D_OUT))


def _bigmm_body(s_ref, ui_ref, uj_ref, o_ref):
    inv = 1.0 / s_ref[0]
    t = jax.lax.dot_general(ui_ref[...], uj_ref[...], (((1,), (1,)), ((), ())),
                            preferred_element_type=jnp.float32)
    o_ref[...] = jax.nn.sigmoid(t * inv)


def _bigmm(u, s, bi=200):
    grid = (N // bi,)
    return pl.pallas_call(
        _bigmm_body,
        grid=grid,
        in_specs=[
            pl.BlockSpec(memory_space=pltpu.SMEM),
            pl.BlockSpec((bi, D_OUT), lambda i: (i, 0)),
            pl.BlockSpec((N, D_OUT), lambda i: (0, 0)),
        ],
        out_specs=pl.BlockSpec((bi, N), lambda i: (i, 0)),
        out_shape=jax.ShapeDtypeStruct((N, N), jnp.float32),
    )(s, u, u)


def kernel(x, edge_index, W, att_src, att_dst, bias):
    h, a_s, a_d = _front(x, W, att_src, att_dst)
    src = edge_index[0]
    dst = edge_index[1]
    asn = a_s[:, 0]
    adn = a_d[:, 0]
    w = jnp.exp(_leaky(asn[src] + adn[dst], 0.2))
    denom = jax.ops.segment_sum(w, dst, num_segments=N)
    agg = jax.ops.segment_sum(w[:, None] * h[src], dst, num_segments=N)
    u, s = _mid(agg, denom.reshape(N, 1), h, a_s, a_d, bias)
    return _bigmm(u, s)


# speedup vs baseline: 26.8048x; 14.8301x over previous
import functools

import jax
import jax.numpy as jnp
from jax import lax
from jax.experimental import pallas as pl
from jax.experimental.pallas import tpu as pltpu
from jax.experimental.pallas import tpu_sc as plsc

N = 10000
E = 320000
D_IN = 128
D_OUT = 256
NQ = 4
DQ = D_OUT // NQ
NSUB = 16
EPW = E // NSUB
SEG = 4000
NSEG = EPW // SEG
BK = 80
NBS = SEG // BK


def _leaky(x, slope):
    return jnp.where(x >= 0, x, slope * x)


def _front_body(x_ref, w_ref, as_ref, ad_ref, h_ref, hs_ref, asn_ref, adn_ref):
    h = jax.lax.dot_general(x_ref[...], w_ref[...], (((1,), (1,)), ((), ())),
                            preferred_element_type=jnp.float32)
    h_ref[...] = h
    for q in range(NQ):
        hs_ref[q * N:(q + 1) * N, :] = h[:, q * DQ:(q + 1) * DQ]
    asn_ref[...] = jax.lax.dot_general(h, as_ref[...], (((1,), (0,)), ((), ())),
                                       preferred_element_type=jnp.float32)
    adn_ref[...] = jax.lax.dot_general(h, ad_ref[...], (((1,), (0,)), ((), ())),
                                       preferred_element_type=jnp.float32)


def _front(x, W, att_src, att_dst):
    return pl.pallas_call(
        _front_body,
        out_shape=(
            jax.ShapeDtypeStruct((N, D_OUT), jnp.float32),
            jax.ShapeDtypeStruct((NQ * N, DQ), jnp.float32),
            jax.ShapeDtypeStruct((N, 1), jnp.float32),
            jax.ShapeDtypeStruct((N, 1), jnp.float32),
        ),
    )(x, W, att_src.reshape(D_OUT, 1), att_dst.reshape(D_OUT, 1))


def _wden_body(src_hbm, dst_hbm, as_hbm, ad_hbm, w_hbm, denp_hbm,
               src_v, dst_v, w_v, as_v, ad_v, den_v, ls0, ls1, ls2, ls3):
    c = lax.axis_index("c")
    s = lax.axis_index("s")
    wid = c * NSUB + s
    zero16 = jnp.zeros((16,), jnp.float32)

    cp0 = pltpu.async_copy(src_hbm.at[wid, 0], src_v, ls0)
    cp1 = pltpu.async_copy(dst_hbm.at[wid], dst_v, ls1)
    cp2 = pltpu.async_copy(as_hbm, as_v, ls2)
    cp3 = pltpu.async_copy(ad_hbm, ad_v, ls3)
    cp0.wait()
    cp1.wait()
    cp2.wait()
    cp3.wait()

    def _zden(i, carry):
        den_v[pl.ds(i * 16, 16)] = zero16
        return carry
    lax.fori_loop(0, N // 16, _zden, 0)

    def _p1(b, carry):
        for gg in range(BK // 16):
            off = b * BK + gg * 16
            sv = src_v[pl.ds(off, 16)]
            dv = dst_v[b, pl.ds(gg * 16, 16)]
            asg = plsc.load_gather(as_v, [sv])
            adg = plsc.load_gather(ad_v, [dv])
            e = asg + adg
            e = jnp.maximum(e, 0.2 * e)
            w = jnp.exp(e)
            w_v[pl.ds(off, 16)] = w
            plsc.addupdate_scatter(den_v, [dv], w)
        return carry
    lax.fori_loop(0, NBA, _p1, 0)

    cp4 = pltpu.async_copy(w_v, w_hbm.at[wid, 0], ls0)
    cp5 = pltpu.async_copy(den_v, denp_hbm.at[wid, 0], ls1)
    cp4.wait()
    cp5.wait()


def _agg_body(src_hbm, dst_hbm, w_hbm, hs_hbm, agg_hbm,
              src_v, dst_v, w_v, rows_v, agg_sh,
              gs0, gs1, gs2, gs3, ss0, ss1, ss2, ss3):
    c = lax.axis_index("c")
    s = lax.axis_index("s")
    cN = c * N
    zero16 = jnp.zeros((16,), jnp.float32)
    gsems = (gs0, gs1, gs2, gs3)
    ssems = (ss0, ss1, ss2, ss3)

    def _zrows(i, carry):
        for j in range(DH // 16):
            rows_v[0, i, pl.ds(j * 16, 16)] = zero16
        return carry
    lax.fori_loop(0, BK, _zrows, 0)

    def _zagg(i, carry):
        t = s + i * NSUB

        @pl.when(t < N // BK)
        def _():
            pltpu.async_copy(rows_v.at[0], agg_sh.at[pl.ds(t * BK, BK)], gs0)
        return carry
    lax.fori_loop(0, 8, _zagg, 0)

    def _zagg_w(i, carry):
        t = s + i * NSUB

        @pl.when(t < N // BK)
        def _():
            pltpu.make_async_copy(rows_v.at[0],
                                  agg_sh.at[pl.ds(t * BK, BK)], gs0).wait()
        return carry
    lax.fori_loop(0, 8, _zagg_w, 0)
    plsc.subcore_barrier()

    def start_gather(b, sl):
        pltpu.async_copy(hs_hbm.at[src_v.at[pl.ds(b * BK, BK)]],
                         rows_v.at[sl], gsems[sl])

    def wait_gather(b, sl):
        pltpu.make_async_copy(hs_hbm.at[src_v.at[pl.ds(b * BK, BK)]],
                              rows_v.at[sl], gsems[sl]).wait()

    def start_scatter(b, sl):
        pltpu.async_copy(rows_v.at[sl], agg_sh.at[dst_v.at[b]],
                         ssems[sl], add=True)

    def wait_scatter(b, sl):
        pltpu.make_async_copy(rows_v.at[sl], agg_sh.at[dst_v.at[b]],
                              ssems[sl]).wait()

    def scale(b, sl):
        @plsc.parallel_loop(0, BK, 1, unroll=4)
        def _sb(e2):
            wv16 = plsc.load_gather(
                w_v, [jnp.full((16,), b * BK + e2, jnp.int32)])
            for j in range(DH // 16):
                rows_v[sl, e2, pl.ds(j * 16, 16)] = (
                    rows_v[sl, e2, pl.ds(j * 16, 16)] * wv16)

    def blk(b, r):
        wait_gather(b, r)
        scale(b, r)
        start_scatter(b, r)
        prev_r = (r + ND - 1) % ND

        @pl.when(b > 0)
        def _():
            wait_scatter(b - 1, prev_r)

        @pl.when(b + ND - 1 < NBS)
        def _():
            start_gather(b + ND - 1, prev_r)

    def _per_seg(seg, carry):
        cpa = pltpu.async_copy(src_hbm.at[s, seg, 0], src_v, gs0)
        cpb = pltpu.async_copy(dst_hbm.at[s, seg], dst_v, gs1)
        cpc = pltpu.async_copy(w_hbm.at[s, seg, 0], w_v, gs2)
        cpa.wait()
        cpb.wait()
        cpc.wait()

        def _adj(i, carry1):
            src_v[pl.ds(i * 16, 16)] = src_v[pl.ds(i * 16, 16)] + cN
            return carry1
        lax.fori_loop(0, SEG // 16, _adj, 0)

        start_gather(0, 0)
        start_gather(1, 1)
        start_gather(2, 2)

        def _p2(bb, carry2):
            for r in range(ND):
                blk(bb * ND + r, r)
            return carry2
        lax.fori_loop(0, NBS // ND, _p2, 0)
        blk(NBS - 1, (NBS - 1) % ND)
        wait_scatter(NBS - 1, (NBS - 1) % ND)
        return carry

    lax.fori_loop(0, NSEG, _per_seg, 0)
    plsc.subcore_barrier()

    def _drain(i, carry):
        t = s + i * NSUB

        @pl.when(t < N // BK)
        def _():
            pltpu.async_copy(agg_sh.at[pl.ds(t * BK, BK)],
                             agg_hbm.at[pl.ds(cN + t * BK, BK)], ss0)
        return carry
    lax.fori_loop(0, 8, _drain, 0)

    def _drain_w(i, carry):
        t = s + i * NSUB

        @pl.when(t < N // BK)
        def _():
            pltpu.make_async_copy(agg_sh.at[pl.ds(t * BK, BK)],
                                  agg_hbm.at[pl.ds(cN + t * BK, BK)],
                                  ss0).wait()
        return carry
    lax.fori_loop(0, 8, _drain_w, 0)


def _edge_phase(src, dst, a_s, a_d, hs):
    wden = pl.kernel(
        _wden_body,
        out_type=(
            jax.ShapeDtypeStruct((NW, 1, EPA), jnp.float32),
            jax.ShapeDtypeStruct((NW, 1, N), jnp.float32),
        ),
        mesh=plsc.VectorSubcoreMesh(core_axis_name="c", subcore_axis_name="s"),
        compiler_params=pltpu.CompilerParams(needs_layout_passes=False),
        scratch_types=[
            pltpu.VMEM((EPA,), jnp.int32),
            pltpu.VMEM((NBA, BK), jnp.int32),
            pltpu.VMEM((EPA,), jnp.float32),
            pltpu.VMEM((N,), jnp.float32),
            pltpu.VMEM((N,), jnp.float32),
            pltpu.VMEM((N,), jnp.float32),
            pltpu.SemaphoreType.DMA,
            pltpu.SemaphoreType.DMA,
            pltpu.SemaphoreType.DMA,
            pltpu.SemaphoreType.DMA,
        ],
    )
    w, denp = wden(src.reshape(NW, 1, EPA), dst.reshape(NW, NBA, BK),
                   a_s, a_d)

    agg_fn = pl.kernel(
        _agg_body,
        out_type=jax.ShapeDtypeStruct((2 * N, DH), jnp.float32),
        mesh=plsc.VectorSubcoreMesh(core_axis_name="c", subcore_axis_name="s"),
        compiler_params=pltpu.CompilerParams(needs_layout_passes=False),
        scratch_types=[
            pltpu.VMEM((SEG,), jnp.int32),
            pltpu.VMEM((NBS, BK), jnp.int32),
            pltpu.VMEM((SEG,), jnp.float32),
            pltpu.VMEM((ND, BK, DH), jnp.float32),
            pltpu.VMEM_SHARED((N, DH), jnp.float32),
            pltpu.SemaphoreType.DMA,
            pltpu.SemaphoreType.DMA,
            pltpu.SemaphoreType.DMA,
            pltpu.SemaphoreType.DMA,
            pltpu.SemaphoreType.DMA,
            pltpu.SemaphoreType.DMA,
            pltpu.SemaphoreType.DMA,
            pltpu.SemaphoreType.DMA,
        ],
    )
    agg = agg_fn(src.reshape(NSUB, NSEG, 1, SEG),
                 dst.reshape(NSUB, NSEG, NBS, BK),
                 w.reshape(NSUB, NSEG, 1, SEG), hs)
    return agg, denp.reshape(NW, N)


def _mid_body(agg_ref, denp_ref, h_ref, as_ref, ad_ref, b_ref, u_ref, s_ref):
    w_self = jnp.exp(_leaky(as_ref[...] + ad_ref[...], 0.2))
    agg = jnp.concatenate([agg_ref[q * N:(q + 1) * N, :] for q in range(NQ)],
                          axis=1)
    agg = agg + w_self * h_ref[...]
    den = jnp.sum(denp_ref[...], axis=0)[:, None] + w_self
    u = _leaky(agg / (den + 1e-16) + b_ref[...], 0.02)
    u_ref[...] = u
    s_ref[0] = jnp.sum(u * u)


def _mid(agg, denp, hs, a_s, a_d, bias):
    return pl.pallas_call(
        _mid_body,
        out_shape=(
            jax.ShapeDtypeStruct((N, D_OUT), jnp.float32),
            jax.ShapeDtypeStruct((1,), jnp.float32),
        ),
        out_specs=(
            pl.BlockSpec(memory_space=pltpu.VMEM),
            pl.BlockSpec(memory_space=pltpu.SMEM),
        ),
    )(agg, denp, hs, a_s, a_d, bias.reshape(1, D_OUT))


def _bigmm_body(s_ref, ui_ref, uj_ref, o_ref):
    inv = 1.0 / s_ref[0]
    t = jax.lax.dot_general(ui_ref[...], uj_ref[...], (((1,), (1,)), ((), ())),
                            preferred_element_type=jnp.float32)
    o_ref[...] = jax.nn.sigmoid(t * inv)


def _bigmm(u, s, bi=400):
    grid = (N // bi,)
    return pl.pallas_call(
        _bigmm_body,
        grid=grid,
        in_specs=[
            pl.BlockSpec(memory_space=pltpu.SMEM),
            pl.BlockSpec((bi, D_OUT), lambda i: (i, 0)),
            pl.BlockSpec((N, D_OUT), lambda i: (0, 0)),
        ],
        out_specs=pl.BlockSpec((bi, N), lambda i: (i, 0)),
        out_shape=jax.ShapeDtypeStruct((N, N), jnp.float32),
    )(s, u, u)


def kernel(x, edge_index, W, att_src, att_dst, bias):
    hs, a_s, a_d = _front(x, W, att_src, att_dst)
    src = edge_index[0]
    dst = edge_index[1]
    agg, denp = _edge_phase(src, dst, a_s[:, 0], a_d[:, 0], hs)
    u, s = _mid(agg, denp, hs, a_s, a_d, bias)
    return _bigmm(u, s)


# speedup vs baseline: 26.8661x; 1.0023x over previous
import functools

import jax
import jax.numpy as jnp
from jax import lax
from jax.experimental import pallas as pl
from jax.experimental.pallas import tpu as pltpu
from jax.experimental.pallas import tpu_sc as plsc

N = 10000
E = 320000
D_IN = 128
D_OUT = 256
NQ = 4
DQ = D_OUT // NQ
NSUB = 16
EPW = E // NSUB
SEG = 4000
NSEG = EPW // SEG
BK = 80
NBS = SEG // BK


def _leaky(x, slope):
    return jnp.where(x >= 0, x, slope * x)


def _front_body(x_ref, w_ref, as_ref, ad_ref, h_ref, hs_ref, asn_ref, adn_ref):
    h = jax.lax.dot_general(x_ref[...], w_ref[...], (((1,), (1,)), ((), ())),
                            preferred_element_type=jnp.float32)
    h_ref[...] = h
    for q in range(NQ):
        hs_ref[q * N:(q + 1) * N, :] = h[:, q * DQ:(q + 1) * DQ]
    asn_ref[...] = jax.lax.dot_general(h, as_ref[...], (((1,), (0,)), ((), ())),
                                       preferred_element_type=jnp.float32)
    adn_ref[...] = jax.lax.dot_general(h, ad_ref[...], (((1,), (0,)), ((), ())),
                                       preferred_element_type=jnp.float32)


def _front(x, W, att_src, att_dst):
    return pl.pallas_call(
        _front_body,
        out_shape=(
            jax.ShapeDtypeStruct((N, D_OUT), jnp.float32),
            jax.ShapeDtypeStruct((NQ * N, DQ), jnp.float32),
            jax.ShapeDtypeStruct((N, 1), jnp.float32),
            jax.ShapeDtypeStruct((N, 1), jnp.float32),
        ),
    )(x, W, att_src.reshape(D_OUT, 1), att_dst.reshape(D_OUT, 1))


def _wden_body(src_hbm, dst_hbm, as_hbm, ad_hbm, w_hbm, denp_hbm,
               src_v, dst_v, w_v, as_v, ad_v, den_v, ls0, ls1, ls2, ls3):
    c = lax.axis_index("c")
    s = lax.axis_index("s")
    wid = c * NSUB + s
    zero16 = jnp.zeros((16,), jnp.float32)

    cp0 = pltpu.async_copy(src_hbm.at[wid, 0], src_v, ls0)
    cp1 = pltpu.async_copy(dst_hbm.at[wid], dst_v, ls1)
    cp2 = pltpu.async_copy(as_hbm, as_v, ls2)
    cp3 = pltpu.async_copy(ad_hbm, ad_v, ls3)
    cp0.wait()
    cp1.wait()
    cp2.wait()
    cp3.wait()

    def _zden(i, carry):
        den_v[pl.ds(i * 16, 16)] = zero16
        return carry
    lax.fori_loop(0, N // 16, _zden, 0)

    def _p1(b, carry):
        for gg in range(BK // 16):
            off = b * BK + gg * 16
            sv = src_v[pl.ds(off, 16)]
            dv = dst_v[b, pl.ds(gg * 16, 16)]
            asg = plsc.load_gather(as_v, [sv])
            adg = plsc.load_gather(ad_v, [dv])
            e = asg + adg
            e = jnp.maximum(e, 0.2 * e)
            w = jnp.exp(e)
            w_v[pl.ds(off, 16)] = w
            plsc.addupdate_scatter(den_v, [dv], w)
        return carry
    lax.fori_loop(0, NBA, _p1, 0)

    cp4 = pltpu.async_copy(w_v, w_hbm.at[wid, 0], ls0)
    cp5 = pltpu.async_copy(den_v, denp_hbm.at[wid, 0], ls1)
    cp4.wait()
    cp5.wait()


def _agg_body(src_hbm, dst_hbm, w_hbm, hs_hbm, agg_hbm,
              src_v, dst_v, w_v, rows_v, agg_sh,
              gs0, gs1, gs2, gs3, ss0, ss1, ss2, ss3):
    c = lax.axis_index("c")
    s = lax.axis_index("s")
    cN = c * N
    zero16 = jnp.zeros((16,), jnp.float32)
    gsems = (gs0, gs1, gs2, gs3)
    ssems = (ss0, ss1, ss2, ss3)

    def _zrows(i, carry):
        for j in range(DH // 16):
            rows_v[0, i, pl.ds(j * 16, 16)] = zero16
        return carry
    lax.fori_loop(0, BK, _zrows, 0)

    def _zagg(i, carry):
        t = s + i * NSUB

        @pl.when(t < N // BK)
        def _():
            pltpu.async_copy(rows_v.at[0], agg_sh.at[pl.ds(t * BK, BK)], gs0)
        return carry
    lax.fori_loop(0, 8, _zagg, 0)

    def _zagg_w(i, carry):
        t = s + i * NSUB

        @pl.when(t < N // BK)
        def _():
            pltpu.make_async_copy(rows_v.at[0],
                                  agg_sh.at[pl.ds(t * BK, BK)], gs0).wait()
        return carry
    lax.fori_loop(0, 8, _zagg_w, 0)
    plsc.subcore_barrier()

    def start_gather(b, sl):
        pltpu.async_copy(hs_hbm.at[src_v.at[pl.ds(b * BK, BK)]],
                         rows_v.at[sl], gsems[sl])

    def wait_gather(b, sl):
        pltpu.make_async_copy(hs_hbm.at[src_v.at[pl.ds(b * BK, BK)]],
                              rows_v.at[sl], gsems[sl]).wait()

    def start_scatter(b, sl):
        pltpu.async_copy(rows_v.at[sl], agg_sh.at[dst_v.at[b]],
                         ssems[sl], add=True)

    def wait_scatter(b, sl):
        pltpu.make_async_copy(rows_v.at[sl], agg_sh.at[dst_v.at[b]],
                              ssems[sl]).wait()

    def scale(b, sl):
        @plsc.parallel_loop(0, BK, 1, unroll=4)
        def _sb(e2):
            wv16 = plsc.load_gather(
                w_v, [jnp.full((16,), b * BK + e2, jnp.int32)])
            for j in range(DH // 16):
                rows_v[sl, e2, pl.ds(j * 16, 16)] = (
                    rows_v[sl, e2, pl.ds(j * 16, 16)] * wv16)

    def blk(b, r):
        wait_gather(b, r)
        scale(b, r)
        start_scatter(b, r)
        prev_r = (r + ND - 1) % ND

        @pl.when(b > 0)
        def _():
            wait_scatter(b - 1, prev_r)

        @pl.when(b + ND - 1 < NBS)
        def _():
            start_gather(b + ND - 1, prev_r)

    def _per_seg(seg, carry):
        cpa = pltpu.async_copy(src_hbm.at[s, seg, 0], src_v, gs0)
        cpb = pltpu.async_copy(dst_hbm.at[s, seg], dst_v, gs1)
        cpc = pltpu.async_copy(w_hbm.at[s, seg, 0], w_v, gs2)
        cpa.wait()
        cpb.wait()
        cpc.wait()

        def _adj(i, carry1):
            src_v[pl.ds(i * 16, 16)] = src_v[pl.ds(i * 16, 16)] + cN
            return carry1
        lax.fori_loop(0, SEG // 16, _adj, 0)

        start_gather(0, 0)
        start_gather(1, 1)
        start_gather(2, 2)

        def _p2(bb, carry2):
            for r in range(ND):
                blk(bb * ND + r, r)
            return carry2
        lax.fori_loop(0, NBS // ND, _p2, 0)
        blk(NBS - 1, (NBS - 1) % ND)
        wait_scatter(NBS - 1, (NBS - 1) % ND)
        return carry

    lax.fori_loop(0, NSEG, _per_seg, 0)
    plsc.subcore_barrier()

    def _drain(i, carry):
        t = s + i * NSUB

        @pl.when(t < N // BK)
        def _():
            pltpu.async_copy(agg_sh.at[pl.ds(t * BK, BK)],
                             agg_hbm.at[pl.ds(cN + t * BK, BK)], ss0)
        return carry
    lax.fori_loop(0, 8, _drain, 0)

    def _drain_w(i, carry):
        t = s + i * NSUB

        @pl.when(t < N // BK)
        def _():
            pltpu.make_async_copy(agg_sh.at[pl.ds(t * BK, BK)],
                                  agg_hbm.at[pl.ds(cN + t * BK, BK)],
                                  ss0).wait()
        return carry
    lax.fori_loop(0, 8, _drain_w, 0)


def _edge_phase(src, dst, a_s, a_d, hs):
    wden = pl.kernel(
        _wden_body,
        out_type=(
            jax.ShapeDtypeStruct((NW, 1, EPA), jnp.float32),
            jax.ShapeDtypeStruct((NW, 1, N), jnp.float32),
        ),
        mesh=plsc.VectorSubcoreMesh(core_axis_name="c", subcore_axis_name="s"),
        compiler_params=pltpu.CompilerParams(needs_layout_passes=False),
        scratch_types=[
            pltpu.VMEM((EPA,), jnp.int32),
            pltpu.VMEM((NBA, BK), jnp.int32),
            pltpu.VMEM((EPA,), jnp.float32),
            pltpu.VMEM((N,), jnp.float32),
            pltpu.VMEM((N,), jnp.float32),
            pltpu.VMEM((N,), jnp.float32),
            pltpu.SemaphoreType.DMA,
            pltpu.SemaphoreType.DMA,
            pltpu.SemaphoreType.DMA,
            pltpu.SemaphoreType.DMA,
        ],
    )
    w, denp = wden(src.reshape(NW, 1, EPA), dst.reshape(NW, NBA, BK),
                   a_s, a_d)

    agg_fn = pl.kernel(
        _agg_body,
        out_type=jax.ShapeDtypeStruct((2 * N, DH), jnp.float32),
        mesh=plsc.VectorSubcoreMesh(core_axis_name="c", subcore_axis_name="s"),
        compiler_params=pltpu.CompilerParams(needs_layout_passes=False),
        scratch_types=[
            pltpu.VMEM((SEG,), jnp.int32),
            pltpu.VMEM((NBS, BK), jnp.int32),
            pltpu.VMEM((SEG,), jnp.float32),
            pltpu.VMEM((ND, BK, DH), jnp.float32),
            pltpu.VMEM_SHARED((N, DH), jnp.float32),
            pltpu.SemaphoreType.DMA,
            pltpu.SemaphoreType.DMA,
            pltpu.SemaphoreType.DMA,
            pltpu.SemaphoreType.DMA,
            pltpu.SemaphoreType.DMA,
            pltpu.SemaphoreType.DMA,
            pltpu.SemaphoreType.DMA,
            pltpu.SemaphoreType.DMA,
        ],
    )
    agg = agg_fn(src.reshape(NSUB, NSEG, 1, SEG),
                 dst.reshape(NSUB, NSEG, NBS, BK),
                 w.reshape(NSUB, NSEG, 1, SEG), hs)
    return agg, denp.reshape(NW, N)


def _mid_body(agg_ref, denp_ref, h_ref, as_ref, ad_ref, b_ref, u_ref, s_ref):
    w_self = jnp.exp(_leaky(as_ref[...] + ad_ref[...], 0.2))
    agg = jnp.concatenate([agg_ref[q * N:(q + 1) * N, :] for q in range(NQ)],
                          axis=1)
    agg = agg + w_self * h_ref[...]
    den = jnp.sum(denp_ref[...], axis=0)[:, None] + w_self
    u = _leaky(agg / (den + 1e-16) + b_ref[...], 0.02)
    u_ref[...] = u
    s_ref[0] = jnp.sum(u * u)


def _mid(agg, denp, hs, a_s, a_d, bias):
    return pl.pallas_call(
        _mid_body,
        out_shape=(
            jax.ShapeDtypeStruct((N, D_OUT), jnp.float32),
            jax.ShapeDtypeStruct((1,), jnp.float32),
        ),
        out_specs=(
            pl.BlockSpec(memory_space=pltpu.VMEM),
            pl.BlockSpec(memory_space=pltpu.SMEM),
        ),
    )(agg, denp, hs, a_s, a_d, bias.reshape(1, D_OUT))


def _bigmm_body(s_ref, ui_ref, uj_ref, o_ref):
    inv = 1.0 / s_ref[0]
    t = jax.lax.dot_general(ui_ref[...], uj_ref[...], (((1,), (1,)), ((), ())),
                            preferred_element_type=jnp.float32)
    o_ref[...] = jax.nn.sigmoid(t * inv)


def _bigmm(u, s, bi=400):
    grid = (N // bi,)
    return pl.pallas_call(
        _bigmm_body,
        grid=grid,
        in_specs=[
            pl.BlockSpec(memory_space=pltpu.SMEM),
            pl.BlockSpec((bi, D_OUT), lambda i: (i, 0)),
            pl.BlockSpec((N, D_OUT), lambda i: (0, 0)),
        ],
        out_specs=pl.BlockSpec((bi, N), lambda i: (i, 0)),
        out_shape=jax.ShapeDtypeStruct((N, N), jnp.float32),
    )(s, u, u)


def kernel(x, edge_index, W, att_src, att_dst, bias):
    a_s, a_d = _logits(x, W, att_src, att_dst)
    hs = _front(x, W)
    src = edge_index[0]
    dst = edge_index[1]
    agg, denp = _edge_phase(src, dst, a_s[:, 0], a_d[:, 0], hs)
    u, s = _mid(agg, denp, hs, a_s, a_d, bias)
    return _bigmm(u, s)
